# baseline (speedup 1.0000x reference)
import functools

import jax
import jax.numpy as jnp
from jax import lax
from jax.experimental import pallas as pl
from jax.experimental.pallas import tpu as pltpu
from jax.experimental.pallas import tpu_sc as plsc

NC = 2
NS = 16
LANES = 16
EB = 80
NROT = 3
DEG_NBUF = 5
DEG_EB = 80
DEGW = 16


def _vsc_mesh():
    return plsc.VectorSubcoreMesh(core_axis_name="c", subcore_axis_name="s")


def _sc_degrees(ei_flat, npad):
    E = ei_flat.shape[0] // 2
    epc = E // NS
    nchunks = epc // DEG_EB
    rps = npad // NS

    nsets = nchunks // DEG_NBUF
    assert nsets % 2 == 0 and nchunks % DEG_NBUF == 0

    @functools.partial(
        pl.kernel,
        out_type=jax.ShapeDtypeStruct((NC, npad, DEGW), jnp.float32),
        mesh=_vsc_mesh(),
        scratch_types=[
            pltpu.VMEM((epc,), jnp.int32),
            pltpu.VMEM((DEG_NBUF, DEG_EB), jnp.int32),
            pltpu.VMEM((DEG_NBUF, DEG_EB), jnp.int32),
            pltpu.VMEM((DEG_EB, DEGW), jnp.float32),
            pltpu.VMEM((64, DEGW), jnp.float32),
            pltpu.VMEM_SHARED((npad, DEGW), jnp.float32),
            pltpu.SemaphoreType.DMA,
            pltpu.SemaphoreType.DMA,
            pltpu.SemaphoreType.DMA,
        ],
    )
    def deg_kernel(ei, out, idx1, idx2a, idx2b, ones_v, z_v, acc,
                   semi, sems_a, sems_b):
        cid = lax.axis_index("c")
        sid = lax.axis_index("s")

        fetch = pltpu.async_copy(ei.at[pl.ds(cid * E + sid * epc, epc)],
                                 idx1, semi)

        @pl.loop(0, DEG_EB)
        def _(k):
            ones_v[k, :] = jnp.ones((LANES,), jnp.float32)

        @pl.loop(0, 64)
        def _(k):
            z_v[k, :] = jnp.zeros((LANES,), jnp.float32)

        @pl.loop(0, rps // 64)
        def _(j):
            pltpu.sync_copy(z_v, acc.at[pl.ds(sid * rps + j * 64, 64)])

        plsc.subcore_barrier()
        fetch.wait()

        def phase(t, idx2, sems):
            @pl.when(t >= 2)
            def _():
                for b in range(DEG_NBUF):
                    pltpu.make_async_copy(
                        ones_v, acc.at[idx2.at[b]], sems).wait()
            for b in range(DEG_NBUF):
                for k in range(DEG_EB // LANES):
                    idx2[b, pl.ds(k * LANES, LANES)] = idx1[
                        pl.ds((t * DEG_NBUF + b) * DEG_EB + k * LANES, LANES)]
            for b in range(DEG_NBUF):
                pltpu.async_copy(ones_v, acc.at[idx2.at[b]], sems, add=True)

        @pl.loop(0, nsets, step=2)
        def _(t0):
            phase(t0, idx2a, sems_a)
            phase(t0 + 1, idx2b, sems_b)

        for idx2, sems in ((idx2a, sems_a), (idx2b, sems_b)):
            for b in range(DEG_NBUF):
                pltpu.make_async_copy(ones_v, acc.at[idx2.at[b]], sems).wait()

        plsc.subcore_barrier()
        pltpu.sync_copy(acc.at[pl.ds(sid * rps, rps)],
                        out.at[cid, pl.ds(sid * rps, rps)])

    return deg_kernel(ei_flat)


def _sc_propagate(h, ei_flat, npad):
    E = ei_flat.shape[0] // 2
    D = h.shape[1]
    epw = E // (NC * NS)
    nchunks = epw // EB
    rps = npad // NS


    @functools.partial(
        pl.kernel,
        out_type=jax.ShapeDtypeStruct((NC, npad, D), jnp.float32),
        mesh=_vsc_mesh(),
        scratch_types=[
            pltpu.VMEM((epw,), jnp.int32),
            pltpu.VMEM((NROT, EB), jnp.int32),
            pltpu.VMEM((NROT, EB, D), jnp.float32),
            pltpu.VMEM((16, D), jnp.float32),
            pltpu.VMEM_SHARED((npad, D), jnp.float32),
            pltpu.SemaphoreType.DMA,
            pltpu.SemaphoreType.DMA,
        ] + [pltpu.SemaphoreType.DMA] * NROT,
    )
    def prop_kernel(h_hbm, ei, out, sidx1, didx2, rows, z_v, acc,
                    semi, semg, *sems):
        cid = lax.axis_index("c")
        sid = lax.axis_index("s")
        ebase = (cid * NS + sid) * epw

        f1 = pltpu.async_copy(ei.at[pl.ds(ebase, epw)], sidx1, semi)

        @pl.loop(0, 16)
        def _(k):
            @pl.loop(0, D // LANES)
            def _(j):
                z_v[k, pl.ds(j * LANES, LANES)] = jnp.zeros((LANES,), jnp.float32)

        @pl.loop(0, rps // 16)
        def _(j):
            pltpu.sync_copy(z_v, acc.at[pl.ds(sid * rps + j * 16, 16)])

        plsc.subcore_barrier()
        f1.wait()

        nmain = (nchunks // NROT) * NROT
        ntail = nchunks - nmain

        def body(t0, nb):
            fd, gd = [], []
            for b in range(nb):
                t = t0 + b
                @pl.when(t >= NROT)
                def _():
                    pltpu.make_async_copy(
                        rows.at[b], acc.at[didx2.at[b]], sems[b]).wait()
                fd.append(pltpu.async_copy(
                    ei.at[pl.ds(E + ebase + t * EB, EB)], didx2.at[b], semi))
                gd.append(pltpu.async_copy(
                    h_hbm.at[sidx1.at[pl.ds(t * EB, EB)]],
                    rows.at[b], semg))
            for b in range(nb):
                fd[b].wait()
            for b in range(nb):
                gd[b].wait()
                pltpu.async_copy(rows.at[b], acc.at[didx2.at[b]],
                                 sems[b], add=True)

        @pl.loop(0, nmain, step=NROT)
        def _(t0):
            body(t0, NROT)

        if ntail:
            body(nmain, ntail)

        for b in range(NROT):
            pltpu.make_async_copy(rows.at[b], acc.at[didx2.at[b]],
                                  sems[b]).wait()

        plsc.subcore_barrier()
        pltpu.sync_copy(acc.at[pl.ds(sid * rps, rps)],
                        out.at[cid, pl.ds(sid * rps, rps)])

    return prop_kernel(h, ei_flat)


def _norm_cols(deg_blk):
    ns = deg_blk[0][:, 0:1]
    nd = deg_blk[1][:, 0:1]
    ns = jnp.where(ns > 0, lax.rsqrt(ns), 0.0)
    nd = jnp.where(nd > 0, lax.rsqrt(nd), 0.0)
    return ns, nd


def _prescale_body(deg_ref, x_ref, o_ref):
    ns, _ = _norm_cols(deg_ref[...])
    o_ref[...] = x_ref[...] * ns


def _tc_prescale(degs, x, npad, R=2000):
    n, D = x.shape
    return pl.pallas_call(
        _prescale_body,
        grid=(n // R,),
        in_specs=[pl.BlockSpec((NC, R, DEGW), lambda i: (0, i, 0)),
                  pl.BlockSpec((R, D), lambda i: (i, 0))],
        out_specs=pl.BlockSpec((R, D), lambda i: (i, 0)),
        out_shape=jax.ShapeDtypeStruct((npad, D), jnp.float32),
    )(degs, x)


def _layer_body(deg_ref, p_ref, w_ref, b_ref, o_ref):
    ns, nd = _norm_cols(deg_ref[...])
    t = (p_ref[0] + p_ref[1]) * nd
    h = jnp.dot(t, w_ref[...], preferred_element_type=jnp.float32) + b_ref[...]
    o_ref[...] = jnp.maximum(h, 0.0) * ns


def _tc_layer(degs, p, W, b, n, R=2000):
    _, npad, D = p.shape
    return pl.pallas_call(
        _layer_body,
        grid=(n // R,),
        in_specs=[pl.BlockSpec((NC, R, DEGW), lambda i: (0, i, 0)),
                  pl.BlockSpec((NC, R, D), lambda i: (0, i, 0)),
                  pl.BlockSpec((D, D), lambda i: (0, 0)),
                  pl.BlockSpec((1, D), lambda i: (0, 0))],
        out_specs=pl.BlockSpec((R, D), lambda i: (i, 0)),
        out_shape=jax.ShapeDtypeStruct((npad, D), jnp.float32),
    )(degs, p, W, b)


def _head_body(deg_ref, p_ref, w3_ref, b3_ref, wf1_ref, bf1_ref, wf2_ref,
               bf2_ref, wo_ref, bo_ref, o_ref, acc_ref, *, n_nodes):
    i = pl.program_id(0)
    _, nd = _norm_cols(deg_ref[...])
    t = (p_ref[0] + p_ref[1]) * nd
    s = jnp.sum(t, axis=0, keepdims=True)

    @pl.when(i == 0)
    def _():
        acc_ref[...] = s

    @pl.when(i > 0)
    def _():
        acc_ref[...] += s

    @pl.when(i == pl.num_programs(0) - 1)
    def _():
        m = acc_ref[...] * (1.0 / n_nodes)
        h3 = jnp.dot(m, w3_ref[...], preferred_element_type=jnp.float32) + b3_ref[...]
        f = jnp.maximum(
            jnp.dot(h3, wf1_ref[...], preferred_element_type=jnp.float32)
            + bf1_ref[...], 0.0)
        f = jnp.maximum(
            jnp.dot(f, wf2_ref[...], preferred_element_type=jnp.float32)
            + bf2_ref[...], 0.0)
        logit = jnp.dot(f, wo_ref[...], preferred_element_type=jnp.float32) + bo_ref[...]
        o_ref[...] = 1.0 / (1.0 + jnp.exp(-logit))


def _tc_head(degs, p, W3, b3, Wf1, bf1, Wf2, bf2, Wout, bout, n_nodes, R=1280):
    _, npad, D = p.shape
    return pl.pallas_call(
        functools.partial(_head_body, n_nodes=n_nodes),
        grid=(npad // R,),
        in_specs=[pl.BlockSpec((NC, R, DEGW), lambda i: (0, i, 0)),
                  pl.BlockSpec((NC, R, D), lambda i: (0, i, 0)),
                  pl.BlockSpec((D, D), lambda i: (0, 0)),
                  pl.BlockSpec((1, D), lambda i: (0, 0)),
                  pl.BlockSpec((D, D), lambda i: (0, 0)),
                  pl.BlockSpec((1, D), lambda i: (0, 0)),
                  pl.BlockSpec((D, D), lambda i: (0, 0)),
                  pl.BlockSpec((1, D), lambda i: (0, 0)),
                  pl.BlockSpec((D, 1), lambda i: (0, 0)),
                  pl.BlockSpec((1, 1), lambda i: (0, 0))],
        out_specs=pl.BlockSpec((1, 1), lambda i: (0, 0)),
        out_shape=jax.ShapeDtypeStruct((1, 1), jnp.float32),
        scratch_shapes=[pltpu.VMEM((1, D), jnp.float32)],
    )(degs, p, W3, b3, Wf1, bf1, Wf2, bf2, Wout, bout)


def kernel(x, edge_index, W1, b1, W2, b2, W3, b3, Wf1, bf1, Wf2, bf2, Wout, bout):
    N, D = x.shape
    npad = ((N + 2047) // 2048) * 2048
    ei_flat = edge_index.reshape(-1)

    degs = _sc_degrees(ei_flat, npad)

    h = _tc_prescale(degs, x, npad)
    for W, b in ((W1, b1), (W2, b2)):
        p = _sc_propagate(h, ei_flat, npad)
        h = _tc_layer(degs, p, W, b.reshape(1, -1), N)
    p = _sc_propagate(h, ei_flat, npad)
    return _tc_head(degs, p, W3, b3.reshape(1, -1), Wf1, bf1.reshape(1, -1),
                    Wf2, bf2.reshape(1, -1), Wout, bout.reshape(1, 1), N)


# speedup vs baseline: 1.0168x; 1.0168x over previous
import functools

import jax
import jax.numpy as jnp
from jax import lax
from jax.experimental import pallas as pl
from jax.experimental.pallas import tpu as pltpu
from jax.experimental.pallas import tpu_sc as plsc

NC = 2
NS = 16
LANES = 16
EB = 80
NROT = 3
DEG_NBUF = 5
DEG_EB = 80
DEGW = 16


def _vsc_mesh():
    return plsc.VectorSubcoreMesh(core_axis_name="c", subcore_axis_name="s")


def _sc_degrees(ei_flat, npad):
    E = ei_flat.shape[0] // 2
    epc = E // NS
    nchunks = epc // DEG_EB
    rps = npad // NS

    nsets = nchunks // DEG_NBUF
    assert nsets % 2 == 0 and nchunks % DEG_NBUF == 0

    @functools.partial(
        pl.kernel,
        out_type=jax.ShapeDtypeStruct((NC, npad, DEGW), jnp.float32),
        mesh=_vsc_mesh(),
        scratch_types=[
            pltpu.VMEM((epc,), jnp.int32),
            pltpu.VMEM((DEG_NBUF, DEG_EB), jnp.int32),
            pltpu.VMEM((DEG_NBUF, DEG_EB), jnp.int32),
            pltpu.VMEM((DEG_EB, DEGW), jnp.float32),
            pltpu.VMEM((64, DEGW), jnp.float32),
            pltpu.VMEM_SHARED((npad, DEGW), jnp.float32),
            pltpu.SemaphoreType.DMA,
            pltpu.SemaphoreType.DMA,
            pltpu.SemaphoreType.DMA,
        ],
    )
    def deg_kernel(ei, out, idx1, idx2a, idx2b, ones_v, z_v, acc,
                   semi, sems_a, sems_b):
        cid = lax.axis_index("c")
        sid = lax.axis_index("s")

        fetch = pltpu.async_copy(ei.at[pl.ds(cid * E + sid * epc, epc)],
                                 idx1, semi)

        @pl.loop(0, DEG_EB)
        def _(k):
            ones_v[k, :] = jnp.ones((LANES,), jnp.float32)

        @pl.loop(0, 64)
        def _(k):
            z_v[k, :] = jnp.zeros((LANES,), jnp.float32)

        @pl.loop(0, rps // 64)
        def _(j):
            pltpu.sync_copy(z_v, acc.at[pl.ds(sid * rps + j * 64, 64)])

        plsc.subcore_barrier()
        fetch.wait()

        def phase(t, idx2, sems):
            @pl.when(t >= 2)
            def _():
                for b in range(DEG_NBUF):
                    pltpu.make_async_copy(
                        ones_v, acc.at[idx2.at[b]], sems).wait()
            for b in range(DEG_NBUF):
                for k in range(DEG_EB // LANES):
                    idx2[b, pl.ds(k * LANES, LANES)] = idx1[
                        pl.ds((t * DEG_NBUF + b) * DEG_EB + k * LANES, LANES)]
            for b in range(DEG_NBUF):
                pltpu.async_copy(ones_v, acc.at[idx2.at[b]], sems, add=True)

        @pl.loop(0, nsets, step=2)
        def _(t0):
            phase(t0, idx2a, sems_a)
            phase(t0 + 1, idx2b, sems_b)

        for idx2, sems in ((idx2a, sems_a), (idx2b, sems_b)):
            for b in range(DEG_NBUF):
                pltpu.make_async_copy(ones_v, acc.at[idx2.at[b]], sems).wait()

        plsc.subcore_barrier()
        pltpu.sync_copy(acc.at[pl.ds(sid * rps, rps)],
                        out.at[cid, pl.ds(sid * rps, rps)])

    return deg_kernel(ei_flat)


def _sc_propagate(h, ei_flat, npad):
    E = ei_flat.shape[0] // 2
    D = h.shape[1]
    epw = E // (NC * NS)
    nchunks = epw // EB
    rps = npad // NS


    @functools.partial(
        pl.kernel,
        out_type=jax.ShapeDtypeStruct((NC, npad, D), jnp.float32),
        mesh=_vsc_mesh(),
        scratch_types=[
            pltpu.VMEM((epw,), jnp.int32),
            pltpu.VMEM((NROT, EB), jnp.int32),
            pltpu.VMEM((NROT, EB, D), jnp.float32),
            pltpu.VMEM((16, D), jnp.float32),
            pltpu.VMEM_SHARED((npad, D), jnp.float32),
            pltpu.SemaphoreType.DMA,
            pltpu.SemaphoreType.DMA,
        ] + [pltpu.SemaphoreType.DMA] * NROT,
    )
    def prop_kernel(h_hbm, ei, out, sidx1, didx2, rows, z_v, acc,
                    semi, semg, *sems):
        cid = lax.axis_index("c")
        sid = lax.axis_index("s")
        ebase = (cid * NS + sid) * epw

        f1 = pltpu.async_copy(ei.at[pl.ds(ebase, epw)], sidx1, semi)

        @pl.loop(0, 16)
        def _(k):
            @pl.loop(0, D // LANES)
            def _(j):
                z_v[k, pl.ds(j * LANES, LANES)] = jnp.zeros((LANES,), jnp.float32)

        @pl.loop(0, rps // 16)
        def _(j):
            pltpu.async_copy(z_v, acc.at[pl.ds(sid * rps + j * 16, 16)], semg)

        @pl.loop(0, rps // 16)
        def _(j):
            pltpu.make_async_copy(
                z_v, acc.at[pl.ds(sid * rps + j * 16, 16)], semg).wait()

        plsc.subcore_barrier()
        f1.wait()

        nmain = (nchunks // NROT) * NROT
        ntail = nchunks - nmain

        def body(t0, nb):
            fd, gd = [], []
            for b in range(nb):
                t = t0 + b
                @pl.when(t >= NROT)
                def _():
                    pltpu.make_async_copy(
                        rows.at[b], acc.at[didx2.at[b]], sems[b]).wait()
                fd.append(pltpu.async_copy(
                    ei.at[pl.ds(E + ebase + t * EB, EB)], didx2.at[b], semi))
                gd.append(pltpu.async_copy(
                    h_hbm.at[sidx1.at[pl.ds(t * EB, EB)]],
                    rows.at[b], semg))
            for b in range(nb):
                fd[b].wait()
            for b in range(nb):
                gd[b].wait()
                pltpu.async_copy(rows.at[b], acc.at[didx2.at[b]],
                                 sems[b], add=True)

        @pl.loop(0, nmain, step=NROT)
        def _(t0):
            body(t0, NROT)

        if ntail:
            body(nmain, ntail)

        for b in range(NROT):
            pltpu.make_async_copy(rows.at[b], acc.at[didx2.at[b]],
                                  sems[b]).wait()

        plsc.subcore_barrier()
        pltpu.sync_copy(acc.at[pl.ds(sid * rps, rps)],
                        out.at[cid, pl.ds(sid * rps, rps)])

    return prop_kernel(h, ei_flat)


def _norm_cols(deg_blk):
    ns = deg_blk[0][:, 0:1]
    nd = deg_blk[1][:, 0:1]
    ns = jnp.where(ns > 0, lax.rsqrt(ns), 0.0)
    nd = jnp.where(nd > 0, lax.rsqrt(nd), 0.0)
    return ns, nd


def _prescale_body(deg_ref, x_ref, o_ref):
    ns, _ = _norm_cols(deg_ref[...])
    o_ref[...] = x_ref[...] * ns


def _tc_prescale(degs, x, npad, R=2000):
    n, D = x.shape
    return pl.pallas_call(
        _prescale_body,
        grid=(n // R,),
        in_specs=[pl.BlockSpec((NC, R, DEGW), lambda i: (0, i, 0)),
                  pl.BlockSpec((R, D), lambda i: (i, 0))],
        out_specs=pl.BlockSpec((R, D), lambda i: (i, 0)),
        out_shape=jax.ShapeDtypeStruct((npad, D), jnp.float32),
    )(degs, x)


def _layer_body(deg_ref, p_ref, w_ref, b_ref, o_ref):
    ns, nd = _norm_cols(deg_ref[...])
    t = (p_ref[0] + p_ref[1]) * nd
    h = jnp.dot(t, w_ref[...], preferred_element_type=jnp.float32) + b_ref[...]
    o_ref[...] = jnp.maximum(h, 0.0) * ns


def _tc_layer(degs, p, W, b, n, R=2000):
    _, npad, D = p.shape
    return pl.pallas_call(
        _layer_body,
        grid=(n // R,),
        in_specs=[pl.BlockSpec((NC, R, DEGW), lambda i: (0, i, 0)),
                  pl.BlockSpec((NC, R, D), lambda i: (0, i, 0)),
                  pl.BlockSpec((D, D), lambda i: (0, 0)),
                  pl.BlockSpec((1, D), lambda i: (0, 0))],
        out_specs=pl.BlockSpec((R, D), lambda i: (i, 0)),
        out_shape=jax.ShapeDtypeStruct((npad, D), jnp.float32),
    )(degs, p, W, b)


def _head_body(deg_ref, p_ref, w3_ref, b3_ref, wf1_ref, bf1_ref, wf2_ref,
               bf2_ref, wo_ref, bo_ref, o_ref, acc_ref, *, n_nodes):
    i = pl.program_id(0)
    _, nd = _norm_cols(deg_ref[...])
    t = (p_ref[0] + p_ref[1]) * nd
    s = jnp.sum(t, axis=0, keepdims=True)

    @pl.when(i == 0)
    def _():
        acc_ref[...] = s

    @pl.when(i > 0)
    def _():
        acc_ref[...] += s

    @pl.when(i == pl.num_programs(0) - 1)
    def _():
        m = acc_ref[...] * (1.0 / n_nodes)
        h3 = jnp.dot(m, w3_ref[...], preferred_element_type=jnp.float32) + b3_ref[...]
        f = jnp.maximum(
            jnp.dot(h3, wf1_ref[...], preferred_element_type=jnp.float32)
            + bf1_ref[...], 0.0)
        f = jnp.maximum(
            jnp.dot(f, wf2_ref[...], preferred_element_type=jnp.float32)
            + bf2_ref[...], 0.0)
        logit = jnp.dot(f, wo_ref[...], preferred_element_type=jnp.float32) + bo_ref[...]
        o_ref[...] = 1.0 / (1.0 + jnp.exp(-logit))


def _tc_head(degs, p, W3, b3, Wf1, bf1, Wf2, bf2, Wout, bout, n_nodes, R=1280):
    _, npad, D = p.shape
    return pl.pallas_call(
        functools.partial(_head_body, n_nodes=n_nodes),
        grid=(npad // R,),
        in_specs=[pl.BlockSpec((NC, R, DEGW), lambda i: (0, i, 0)),
                  pl.BlockSpec((NC, R, D), lambda i: (0, i, 0)),
                  pl.BlockSpec((D, D), lambda i: (0, 0)),
                  pl.BlockSpec((1, D), lambda i: (0, 0)),
                  pl.BlockSpec((D, D), lambda i: (0, 0)),
                  pl.BlockSpec((1, D), lambda i: (0, 0)),
                  pl.BlockSpec((D, D), lambda i: (0, 0)),
                  pl.BlockSpec((1, D), lambda i: (0, 0)),
                  pl.BlockSpec((D, 1), lambda i: (0, 0)),
                  pl.BlockSpec((1, 1), lambda i: (0, 0))],
        out_specs=pl.BlockSpec((1, 1), lambda i: (0, 0)),
        out_shape=jax.ShapeDtypeStruct((1, 1), jnp.float32),
        scratch_shapes=[pltpu.VMEM((1, D), jnp.float32)],
    )(degs, p, W3, b3, Wf1, bf1, Wf2, bf2, Wout, bout)


def kernel(x, edge_index, W1, b1, W2, b2, W3, b3, Wf1, bf1, Wf2, bf2, Wout, bout):
    N, D = x.shape
    npad = ((N + 2047) // 2048) * 2048
    ei_flat = edge_index.reshape(-1)

    degs = _sc_degrees(ei_flat, npad)

    h = _tc_prescale(degs, x, npad)
    for W, b in ((W1, b1), (W2, b2)):
        p = _sc_propagate(h, ei_flat, npad)
        h = _tc_layer(degs, p, W, b.reshape(1, -1), N)
    p = _sc_propagate(h, ei_flat, npad)
    return _tc_head(degs, p, W3, b3.reshape(1, -1), Wf1, bf1.reshape(1, -1),
                    Wf2, bf2.reshape(1, -1), Wout, bout.reshape(1, 1), N)


# speedup vs baseline: 1.0172x; 1.0003x over previous
import functools

import jax
import jax.numpy as jnp
from jax import lax
from jax.experimental import pallas as pl
from jax.experimental.pallas import tpu as pltpu
from jax.experimental.pallas import tpu_sc as plsc

NC = 2
NS = 16
LANES = 16
EB = 80
NROT = 3
DEG_NBUF = 5
DEG_EB = 80
DEGW = 16


def _vsc_mesh():
    return plsc.VectorSubcoreMesh(core_axis_name="c", subcore_axis_name="s")


def _sc_degrees(ei_flat, npad):
    E = ei_flat.shape[0] // 2
    epc = E // NS
    nchunks = epc // DEG_EB
    rps = npad // NS

    nsets = nchunks // DEG_NBUF
    assert nsets % 2 == 0 and nchunks % DEG_NBUF == 0

    @functools.partial(
        pl.kernel,
        out_type=jax.ShapeDtypeStruct((NC, npad, DEGW), jnp.float32),
        mesh=_vsc_mesh(),
        scratch_types=[
            pltpu.VMEM((epc,), jnp.int32),
            pltpu.VMEM((DEG_NBUF, DEG_EB), jnp.int32),
            pltpu.VMEM((DEG_NBUF, DEG_EB), jnp.int32),
            pltpu.VMEM((DEG_EB, DEGW), jnp.float32),
            pltpu.VMEM((64, DEGW), jnp.float32),
            pltpu.VMEM_SHARED((npad, DEGW), jnp.float32),
            pltpu.SemaphoreType.DMA,
            pltpu.SemaphoreType.DMA,
            pltpu.SemaphoreType.DMA,
        ],
    )
    def deg_kernel(ei, out, idx1, idx2a, idx2b, ones_v, z_v, acc,
                   semi, sems_a, sems_b):
        cid = lax.axis_index("c")
        sid = lax.axis_index("s")

        fetch = pltpu.async_copy(ei.at[pl.ds(cid * E + sid * epc, epc)],
                                 idx1, semi)

        @pl.loop(0, DEG_EB)
        def _(k):
            ones_v[k, :] = jnp.ones((LANES,), jnp.float32)

        @pl.loop(0, 64)
        def _(k):
            z_v[k, :] = jnp.zeros((LANES,), jnp.float32)

        @pl.loop(0, rps // 64)
        def _(j):
            pltpu.async_copy(z_v, acc.at[pl.ds(sid * rps + j * 64, 64)],
                             sems_a)

        @pl.loop(0, rps // 64)
        def _(j):
            pltpu.make_async_copy(
                z_v, acc.at[pl.ds(sid * rps + j * 64, 64)], sems_a).wait()

        plsc.subcore_barrier()
        fetch.wait()

        def phase(t, idx2, sems):
            @pl.when(t >= 2)
            def _():
                for b in range(DEG_NBUF):
                    pltpu.make_async_copy(
                        ones_v, acc.at[idx2.at[b]], sems).wait()
            for b in range(DEG_NBUF):
                for k in range(DEG_EB // LANES):
                    idx2[b, pl.ds(k * LANES, LANES)] = idx1[
                        pl.ds((t * DEG_NBUF + b) * DEG_EB + k * LANES, LANES)]
            for b in range(DEG_NBUF):
                pltpu.async_copy(ones_v, acc.at[idx2.at[b]], sems, add=True)

        @pl.loop(0, nsets, step=2)
        def _(t0):
            phase(t0, idx2a, sems_a)
            phase(t0 + 1, idx2b, sems_b)

        for idx2, sems in ((idx2a, sems_a), (idx2b, sems_b)):
            for b in range(DEG_NBUF):
                pltpu.make_async_copy(ones_v, acc.at[idx2.at[b]], sems).wait()

        plsc.subcore_barrier()
        pltpu.sync_copy(acc.at[pl.ds(sid * rps, rps)],
                        out.at[cid, pl.ds(sid * rps, rps)])

    return deg_kernel(ei_flat)


def _sc_propagate(h, ei_flat, npad):
    E = ei_flat.shape[0] // 2
    D = h.shape[1]
    epw = E // (NC * NS)
    nchunks = epw // EB
    rps = npad // NS


    @functools.partial(
        pl.kernel,
        out_type=jax.ShapeDtypeStruct((NC, npad, D), jnp.float32),
        mesh=_vsc_mesh(),
        scratch_types=[
            pltpu.VMEM((epw,), jnp.int32),
            pltpu.VMEM((NROT, EB), jnp.int32),
            pltpu.VMEM((NROT, EB, D), jnp.float32),
            pltpu.VMEM((16, D), jnp.float32),
            pltpu.VMEM_SHARED((npad, D), jnp.float32),
            pltpu.SemaphoreType.DMA,
            pltpu.SemaphoreType.DMA,
        ] + [pltpu.SemaphoreType.DMA] * NROT,
    )
    def prop_kernel(h_hbm, ei, out, sidx1, didx2, rows, z_v, acc,
                    semi, semg, *sems):
        cid = lax.axis_index("c")
        sid = lax.axis_index("s")
        ebase = (cid * NS + sid) * epw

        f1 = pltpu.async_copy(ei.at[pl.ds(ebase, epw)], sidx1, semi)

        @pl.loop(0, 16)
        def _(k):
            @pl.loop(0, D // LANES)
            def _(j):
                z_v[k, pl.ds(j * LANES, LANES)] = jnp.zeros((LANES,), jnp.float32)

        @pl.loop(0, rps // 16)
        def _(j):
            pltpu.async_copy(z_v, acc.at[pl.ds(sid * rps + j * 16, 16)], semg)

        @pl.loop(0, rps // 16)
        def _(j):
            pltpu.make_async_copy(
                z_v, acc.at[pl.ds(sid * rps + j * 16, 16)], semg).wait()

        plsc.subcore_barrier()
        f1.wait()

        nmain = (nchunks // NROT) * NROT
        ntail = nchunks - nmain

        def body(t0, nb):
            fd, gd = [], []
            for b in range(nb):
                t = t0 + b
                @pl.when(t >= NROT)
                def _():
                    pltpu.make_async_copy(
                        rows.at[b], acc.at[didx2.at[b]], sems[b]).wait()
                fd.append(pltpu.async_copy(
                    ei.at[pl.ds(E + ebase + t * EB, EB)], didx2.at[b], semi))
                gd.append(pltpu.async_copy(
                    h_hbm.at[sidx1.at[pl.ds(t * EB, EB)]],
                    rows.at[b], semg))
            for b in range(nb):
                fd[b].wait()
            for b in range(nb):
                gd[b].wait()
                pltpu.async_copy(rows.at[b], acc.at[didx2.at[b]],
                                 sems[b], add=True)

        @pl.loop(0, nmain, step=NROT)
        def _(t0):
            body(t0, NROT)

        if ntail:
            body(nmain, ntail)

        for b in range(NROT):
            pltpu.make_async_copy(rows.at[b], acc.at[didx2.at[b]],
                                  sems[b]).wait()

        plsc.subcore_barrier()
        pltpu.sync_copy(acc.at[pl.ds(sid * rps, rps)],
                        out.at[cid, pl.ds(sid * rps, rps)])

    return prop_kernel(h, ei_flat)


def _norm_cols(deg_blk):
    ns = deg_blk[0][:, 0:1]
    nd = deg_blk[1][:, 0:1]
    ns = jnp.where(ns > 0, lax.rsqrt(ns), 0.0)
    nd = jnp.where(nd > 0, lax.rsqrt(nd), 0.0)
    return ns, nd


def _prescale_body(deg_ref, x_ref, o_ref):
    ns, _ = _norm_cols(deg_ref[...])
    o_ref[...] = x_ref[...] * ns


def _tc_prescale(degs, x, npad, R=2000):
    n, D = x.shape
    return pl.pallas_call(
        _prescale_body,
        grid=(n // R,),
        in_specs=[pl.BlockSpec((NC, R, DEGW), lambda i: (0, i, 0)),
                  pl.BlockSpec((R, D), lambda i: (i, 0))],
        out_specs=pl.BlockSpec((R, D), lambda i: (i, 0)),
        out_shape=jax.ShapeDtypeStruct((npad, D), jnp.float32),
    )(degs, x)


def _layer_body(deg_ref, p_ref, w_ref, b_ref, o_ref):
    ns, nd = _norm_cols(deg_ref[...])
    t = (p_ref[0] + p_ref[1]) * nd
    h = jnp.dot(t, w_ref[...], preferred_element_type=jnp.float32) + b_ref[...]
    o_ref[...] = jnp.maximum(h, 0.0) * ns


def _tc_layer(degs, p, W, b, n, R=2000):
    _, npad, D = p.shape
    return pl.pallas_call(
        _layer_body,
        grid=(n // R,),
        in_specs=[pl.BlockSpec((NC, R, DEGW), lambda i: (0, i, 0)),
                  pl.BlockSpec((NC, R, D), lambda i: (0, i, 0)),
                  pl.BlockSpec((D, D), lambda i: (0, 0)),
                  pl.BlockSpec((1, D), lambda i: (0, 0))],
        out_specs=pl.BlockSpec((R, D), lambda i: (i, 0)),
        out_shape=jax.ShapeDtypeStruct((npad, D), jnp.float32),
    )(degs, p, W, b)


def _head_body(deg_ref, p_ref, w3_ref, b3_ref, wf1_ref, bf1_ref, wf2_ref,
               bf2_ref, wo_ref, bo_ref, o_ref, acc_ref, *, n_nodes):
    i = pl.program_id(0)
    _, nd = _norm_cols(deg_ref[...])
    t = (p_ref[0] + p_ref[1]) * nd
    s = jnp.sum(t, axis=0, keepdims=True)

    @pl.when(i == 0)
    def _():
        acc_ref[...] = s

    @pl.when(i > 0)
    def _():
        acc_ref[...] += s

    @pl.when(i == pl.num_programs(0) - 1)
    def _():
        m = acc_ref[...] * (1.0 / n_nodes)
        h3 = jnp.dot(m, w3_ref[...], preferred_element_type=jnp.float32) + b3_ref[...]
        f = jnp.maximum(
            jnp.dot(h3, wf1_ref[...], preferred_element_type=jnp.float32)
            + bf1_ref[...], 0.0)
        f = jnp.maximum(
            jnp.dot(f, wf2_ref[...], preferred_element_type=jnp.float32)
            + bf2_ref[...], 0.0)
        logit = jnp.dot(f, wo_ref[...], preferred_element_type=jnp.float32) + bo_ref[...]
        o_ref[...] = 1.0 / (1.0 + jnp.exp(-logit))


def _tc_head(degs, p, W3, b3, Wf1, bf1, Wf2, bf2, Wout, bout, n_nodes, R=1280):
    _, npad, D = p.shape
    return pl.pallas_call(
        functools.partial(_head_body, n_nodes=n_nodes),
        grid=(npad // R,),
        in_specs=[pl.BlockSpec((NC, R, DEGW), lambda i: (0, i, 0)),
                  pl.BlockSpec((NC, R, D), lambda i: (0, i, 0)),
                  pl.BlockSpec((D, D), lambda i: (0, 0)),
                  pl.BlockSpec((1, D), lambda i: (0, 0)),
                  pl.BlockSpec((D, D), lambda i: (0, 0)),
                  pl.BlockSpec((1, D), lambda i: (0, 0)),
                  pl.BlockSpec((D, D), lambda i: (0, 0)),
                  pl.BlockSpec((1, D), lambda i: (0, 0)),
                  pl.BlockSpec((D, 1), lambda i: (0, 0)),
                  pl.BlockSpec((1, 1), lambda i: (0, 0))],
        out_specs=pl.BlockSpec((1, 1), lambda i: (0, 0)),
        out_shape=jax.ShapeDtypeStruct((1, 1), jnp.float32),
        scratch_shapes=[pltpu.VMEM((1, D), jnp.float32)],
    )(degs, p, W3, b3, Wf1, bf1, Wf2, bf2, Wout, bout)


def kernel(x, edge_index, W1, b1, W2, b2, W3, b3, Wf1, bf1, Wf2, bf2, Wout, bout):
    N, D = x.shape
    npad = ((N + 2047) // 2048) * 2048
    ei_flat = edge_index.reshape(-1)

    degs = _sc_degrees(ei_flat, npad)

    h = _tc_prescale(degs, x, npad)
    for W, b in ((W1, b1), (W2, b2)):
        p = _sc_propagate(h, ei_flat, npad)
        h = _tc_layer(degs, p, W, b.reshape(1, -1), N)
    p = _sc_propagate(h, ei_flat, npad)
    return _tc_head(degs, p, W3, b3.reshape(1, -1), Wf1, bf1.reshape(1, -1),
                    Wf2, bf2.reshape(1, -1), Wout, bout.reshape(1, 1), N)


# speedup vs baseline: 1.0216x; 1.0044x over previous
import functools

import jax
import jax.numpy as jnp
from jax import lax
from jax.experimental import pallas as pl
from jax.experimental.pallas import tpu as pltpu
from jax.experimental.pallas import tpu_sc as plsc

NC = 2
NS = 16
LANES = 16
EB = 40
NROT = 2
DEG_NBUF = 5
DEG_EB = 80
DEGW = 16


def _vsc_mesh():
    return plsc.VectorSubcoreMesh(core_axis_name="c", subcore_axis_name="s")


def _sc_degrees(ei_flat, npad):
    E = ei_flat.shape[0] // 2
    epc = E // NS
    nchunks = epc // DEG_EB
    rps = npad // NS

    nsets = nchunks // DEG_NBUF
    assert nsets % 2 == 0 and nchunks % DEG_NBUF == 0

    @functools.partial(
        pl.kernel,
        out_type=jax.ShapeDtypeStruct((NC, npad, DEGW), jnp.float32),
        mesh=_vsc_mesh(),
        scratch_types=[
            pltpu.VMEM((epc,), jnp.int32),
            pltpu.VMEM((DEG_NBUF, DEG_EB), jnp.int32),
            pltpu.VMEM((DEG_NBUF, DEG_EB), jnp.int32),
            pltpu.VMEM((DEG_EB, DEGW), jnp.float32),
            pltpu.VMEM((64, DEGW), jnp.float32),
            pltpu.VMEM_SHARED((npad, DEGW), jnp.float32),
            pltpu.SemaphoreType.DMA,
            pltpu.SemaphoreType.DMA,
            pltpu.SemaphoreType.DMA,
        ],
    )
    def deg_kernel(ei, out, idx1, idx2a, idx2b, ones_v, z_v, acc,
                   semi, sems_a, sems_b):
        cid = lax.axis_index("c")
        sid = lax.axis_index("s")

        fetch = pltpu.async_copy(ei.at[pl.ds(cid * E + sid * epc, epc)],
                                 idx1, semi)

        @pl.loop(0, DEG_EB)
        def _(k):
            ones_v[k, :] = jnp.ones((LANES,), jnp.float32)

        @pl.loop(0, 64)
        def _(k):
            z_v[k, :] = jnp.zeros((LANES,), jnp.float32)

        @pl.loop(0, rps // 64)
        def _(j):
            pltpu.async_copy(z_v, acc.at[pl.ds(sid * rps + j * 64, 64)],
                             sems_a)

        @pl.loop(0, rps // 64)
        def _(j):
            pltpu.make_async_copy(
                z_v, acc.at[pl.ds(sid * rps + j * 64, 64)], sems_a).wait()

        plsc.subcore_barrier()
        fetch.wait()

        def phase(t, idx2, sems):
            @pl.when(t >= 2)
            def _():
                for b in range(DEG_NBUF):
                    pltpu.make_async_copy(
                        ones_v, acc.at[idx2.at[b]], sems).wait()
            for b in range(DEG_NBUF):
                for k in range(DEG_EB // LANES):
                    idx2[b, pl.ds(k * LANES, LANES)] = idx1[
                        pl.ds((t * DEG_NBUF + b) * DEG_EB + k * LANES, LANES)]
            for b in range(DEG_NBUF):
                pltpu.async_copy(ones_v, acc.at[idx2.at[b]], sems, add=True)

        @pl.loop(0, nsets, step=2)
        def _(t0):
            phase(t0, idx2a, sems_a)
            phase(t0 + 1, idx2b, sems_b)

        for idx2, sems in ((idx2a, sems_a), (idx2b, sems_b)):
            for b in range(DEG_NBUF):
                pltpu.make_async_copy(ones_v, acc.at[idx2.at[b]], sems).wait()

        plsc.subcore_barrier()
        pltpu.sync_copy(acc.at[pl.ds(sid * rps, rps)],
                        out.at[cid, pl.ds(sid * rps, rps)])

    return deg_kernel(ei_flat)


def _sc_propagate(h, ei_flat, npad):
    E = ei_flat.shape[0] // 2
    D = h.shape[1]
    epw = E // (NC * NS)
    nchunks = epw // EB
    rps = npad // NS


    @functools.partial(
        pl.kernel,
        out_type=jax.ShapeDtypeStruct((NC, npad, D), jnp.float32),
        mesh=_vsc_mesh(),
        scratch_types=[
            pltpu.VMEM((epw,), jnp.int32),
            pltpu.VMEM((NROT, EB), jnp.int32),
            pltpu.VMEM((NROT, EB), jnp.int32),
            pltpu.VMEM((NROT, EB, D), jnp.float32),
            pltpu.VMEM((NROT, EB, D), jnp.float32),
            pltpu.VMEM((16, D), jnp.float32),
            pltpu.VMEM_SHARED((npad, D), jnp.float32),
        ] + [pltpu.SemaphoreType.DMA] * 6,
    )
    def prop_kernel(h_hbm, ei, out, sidx1, didx_a, didx_b, rows_a, rows_b,
                    z_v, acc, semi_a, semi_b, semg_a, semg_b, sems_a, sems_b):
        cid = lax.axis_index("c")
        sid = lax.axis_index("s")
        ebase = (cid * NS + sid) * epw
        nbodies = nchunks // NROT
        seta = (didx_a, rows_a, semi_a, semg_a, sems_a)
        setb = (didx_b, rows_b, semi_b, semg_b, sems_b)

        f1 = pltpu.async_copy(ei.at[pl.ds(ebase, epw)], sidx1, semi_a)

        @pl.loop(0, 16)
        def _(k):
            @pl.loop(0, D // LANES)
            def _(j):
                z_v[k, pl.ds(j * LANES, LANES)] = jnp.zeros((LANES,), jnp.float32)

        @pl.loop(0, rps // 16)
        def _(j):
            pltpu.async_copy(z_v, acc.at[pl.ds(sid * rps + j * 16, 16)], semg_a)

        @pl.loop(0, rps // 16)
        def _(j):
            pltpu.make_async_copy(
                z_v, acc.at[pl.ds(sid * rps + j * 16, 16)], semg_a).wait()

        plsc.subcore_barrier()
        f1.wait()

        def fire_set(k, st):
            didx2, rows, semi, semg, _ = st
            for b in range(NROT):
                t = k * NROT + b
                pltpu.async_copy(ei.at[pl.ds(E + ebase + t * EB, EB)],
                                 didx2.at[b], semi)
                pltpu.async_copy(h_hbm.at[sidx1.at[pl.ds(t * EB, EB)]],
                                 rows.at[b], semg)

        def drain_scatters(st):
            didx2, rows, _, _, sems = st
            for b in range(NROT):
                pltpu.make_async_copy(rows.at[b], acc.at[didx2.at[b]],
                                      sems).wait()

        def half(k, cur, nxt):
            didx2, rows, semi, semg, sems = cur

            @pl.when(k >= 1)
            def _():
                drain_scatters(nxt)

            @pl.when(k + 1 < nbodies)
            def _():
                fire_set(k + 1, nxt)

            for b in range(NROT):
                t = k * NROT + b
                pltpu.make_async_copy(ei.at[pl.ds(E + ebase + t * EB, EB)],
                                      didx2.at[b], semi).wait()
            for b in range(NROT):
                t = k * NROT + b
                pltpu.make_async_copy(
                    h_hbm.at[sidx1.at[pl.ds(t * EB, EB)]],
                    rows.at[b], semg).wait()
                pltpu.async_copy(rows.at[b], acc.at[didx2.at[b]], sems,
                                 add=True)

        fire_set(0, seta)

        @pl.loop(0, nbodies - (nbodies % 2), step=2)
        def _(k0):
            half(k0, seta, setb)
            half(k0 + 1, setb, seta)

        if nbodies % 2:
            half(nbodies - 1, seta, setb)
            drain_scatters(seta)
        else:
            drain_scatters(setb)
            drain_scatters(seta)

        plsc.subcore_barrier()
        pltpu.sync_copy(acc.at[pl.ds(sid * rps, rps)],
                        out.at[cid, pl.ds(sid * rps, rps)])

    return prop_kernel(h, ei_flat)


def _norm_cols(deg_blk):
    ns = deg_blk[0][:, 0:1]
    nd = deg_blk[1][:, 0:1]
    ns = jnp.where(ns > 0, lax.rsqrt(ns), 0.0)
    nd = jnp.where(nd > 0, lax.rsqrt(nd), 0.0)
    return ns, nd


def _prescale_body(deg_ref, x_ref, o_ref):
    ns, _ = _norm_cols(deg_ref[...])
    o_ref[...] = x_ref[...] * ns


def _tc_prescale(degs, x, npad, R=2000):
    n, D = x.shape
    return pl.pallas_call(
        _prescale_body,
        grid=(n // R,),
        in_specs=[pl.BlockSpec((NC, R, DEGW), lambda i: (0, i, 0)),
                  pl.BlockSpec((R, D), lambda i: (i, 0))],
        out_specs=pl.BlockSpec((R, D), lambda i: (i, 0)),
        out_shape=jax.ShapeDtypeStruct((npad, D), jnp.float32),
    )(degs, x)


def _layer_body(deg_ref, p_ref, w_ref, b_ref, o_ref):
    ns, nd = _norm_cols(deg_ref[...])
    t = (p_ref[0] + p_ref[1]) * nd
    h = jnp.dot(t, w_ref[...], preferred_element_type=jnp.float32) + b_ref[...]
    o_ref[...] = jnp.maximum(h, 0.0) * ns


def _tc_layer(degs, p, W, b, n, R=2000):
    _, npad, D = p.shape
    return pl.pallas_call(
        _layer_body,
        grid=(n // R,),
        in_specs=[pl.BlockSpec((NC, R, DEGW), lambda i: (0, i, 0)),
                  pl.BlockSpec((NC, R, D), lambda i: (0, i, 0)),
                  pl.BlockSpec((D, D), lambda i: (0, 0)),
                  pl.BlockSpec((1, D), lambda i: (0, 0))],
        out_specs=pl.BlockSpec((R, D), lambda i: (i, 0)),
        out_shape=jax.ShapeDtypeStruct((npad, D), jnp.float32),
    )(degs, p, W, b)


def _head_body(deg_ref, p_ref, w3_ref, b3_ref, wf1_ref, bf1_ref, wf2_ref,
               bf2_ref, wo_ref, bo_ref, o_ref, acc_ref, *, n_nodes):
    i = pl.program_id(0)
    _, nd = _norm_cols(deg_ref[...])
    t = (p_ref[0] + p_ref[1]) * nd
    s = jnp.sum(t, axis=0, keepdims=True)

    @pl.when(i == 0)
    def _():
        acc_ref[...] = s

    @pl.when(i > 0)
    def _():
        acc_ref[...] += s

    @pl.when(i == pl.num_programs(0) - 1)
    def _():
        m = acc_ref[...] * (1.0 / n_nodes)
        h3 = jnp.dot(m, w3_ref[...], preferred_element_type=jnp.float32) + b3_ref[...]
        f = jnp.maximum(
            jnp.dot(h3, wf1_ref[...], preferred_element_type=jnp.float32)
            + bf1_ref[...], 0.0)
        f = jnp.maximum(
            jnp.dot(f, wf2_ref[...], preferred_element_type=jnp.float32)
            + bf2_ref[...], 0.0)
        logit = jnp.dot(f, wo_ref[...], preferred_element_type=jnp.float32) + bo_ref[...]
        o_ref[...] = 1.0 / (1.0 + jnp.exp(-logit))


def _tc_head(degs, p, W3, b3, Wf1, bf1, Wf2, bf2, Wout, bout, n_nodes, R=1280):
    _, npad, D = p.shape
    return pl.pallas_call(
        functools.partial(_head_body, n_nodes=n_nodes),
        grid=(npad // R,),
        in_specs=[pl.BlockSpec((NC, R, DEGW), lambda i: (0, i, 0)),
                  pl.BlockSpec((NC, R, D), lambda i: (0, i, 0)),
                  pl.BlockSpec((D, D), lambda i: (0, 0)),
                  pl.BlockSpec((1, D), lambda i: (0, 0)),
                  pl.BlockSpec((D, D), lambda i: (0, 0)),
                  pl.BlockSpec((1, D), lambda i: (0, 0)),
                  pl.BlockSpec((D, D), lambda i: (0, 0)),
                  pl.BlockSpec((1, D), lambda i: (0, 0)),
                  pl.BlockSpec((D, 1), lambda i: (0, 0)),
                  pl.BlockSpec((1, 1), lambda i: (0, 0))],
        out_specs=pl.BlockSpec((1, 1), lambda i: (0, 0)),
        out_shape=jax.ShapeDtypeStruct((1, 1), jnp.float32),
        scratch_shapes=[pltpu.VMEM((1, D), jnp.float32)],
    )(degs, p, W3, b3, Wf1, bf1, Wf2, bf2, Wout, bout)


def kernel(x, edge_index, W1, b1, W2, b2, W3, b3, Wf1, bf1, Wf2, bf2, Wout, bout):
    N, D = x.shape
    npad = ((N + 2047) // 2048) * 2048
    ei_flat = edge_index.reshape(-1)

    degs = _sc_degrees(ei_flat, npad)

    h = _tc_prescale(degs, x, npad)
    for W, b in ((W1, b1), (W2, b2)):
        p = _sc_propagate(h, ei_flat, npad)
        h = _tc_layer(degs, p, W, b.reshape(1, -1), N)
    p = _sc_propagate(h, ei_flat, npad)
    return _tc_head(degs, p, W3, b3.reshape(1, -1), Wf1, bf1.reshape(1, -1),
                    Wf2, bf2.reshape(1, -1), Wout, bout.reshape(1, 1), N)
